# baseline (speedup 1.0000x reference)
import jax
import jax.numpy as jnp
from jax import lax
from jax.experimental import pallas as pl
from jax.experimental.pallas import tpu as pltpu
from jax.experimental.pallas import tpu_sc as plsc

ROWS, COLS = 4096, 200
EMBED = 64
VOCAB = 1000000
NC, NS = 2, 16
NW = NC * NS
IBLK = ROWS // NW


def _body(idx_hbm, tab_hbm, out_hbm, idx_v, pidx_v, qoff_v, rows_v, blk_v,
          gsem0, gsem1, osem0, osem1):
    w = lax.axis_index("s") * NC + lax.axis_index("c")
    gsems = (gsem0, gsem1)
    osems = (osem0, osem1)
    lanes = lax.iota(jnp.int32, 16)

    pltpu.sync_copy(idx_hbm.at[:, pl.ds(pl.multiple_of(w * IBLK, IBLK), IBLK)],
                    idx_v)

    def prep(j, b):
        for g in range(IBLK // 16):
            t = idx_v[j, pl.ds(16 * g, 16)]
            pidx_v[b, pl.ds(16 * g, 16)] = lax.shift_right_logical(t, 1)
            qoff_v[b, pl.ds(16 * g, 16)] = (t & 1) * EMBED

    def start_gather(b):
        pltpu.async_copy(tab_hbm.at[pidx_v.at[b]], rows_v.at[b], gsems[b])

    prep(0, 0)
    start_gather(0)

    def step(i, carry):
        for b in range(2):
            j = i * 2 + b

            @pl.when(j < COLS - 1)
            def _():
                prep(j + 1, 1 - b)
                start_gather(1 - b)

            pltpu.make_async_copy(tab_hbm.at[pl.ds(0, IBLK)],
                                  rows_v.at[b], gsems[b]).wait()
            @pl.when(j >= 2)
            def _():
                pltpu.make_async_copy(blk_v.at[b],
                                      out_hbm.at[0, :, 0, :, :],
                                      osems[b]).wait()

            for g in range(IBLK // 16):
                il = lanes + (16 * g)
                qv = qoff_v[b, pl.ds(16 * g, 16)]
                for e in range(EMBED):
                    v = plsc.load_gather(rows_v.at[b], [il, qv + e])
                    blk_v[b, e // 8, e % 8, pl.ds(16 * g, 16)] = v

            pltpu.async_copy(blk_v.at[b], out_hbm.at[j, :, w, :, :], osems[b])
        return carry

    lax.fori_loop(0, COLS // 2, step, 0, unroll=False)

    for b in range(2):
        pltpu.make_async_copy(blk_v.at[b], out_hbm.at[0, :, 0, :, :],
                              osems[b]).wait()


@jax.jit
def _embed(idx_t, tab_pairs):
    mesh = plsc.VectorSubcoreMesh(core_axis_name="c", subcore_axis_name="s")
    fn = pl.kernel(
        _body,
        out_type=jax.ShapeDtypeStruct((COLS, 8, NW, 8, 128), jnp.float32),
        mesh=mesh,
        scratch_types=[
            pltpu.VMEM((COLS, IBLK), jnp.int32),
            pltpu.VMEM((2, IBLK), jnp.int32),
            pltpu.VMEM((2, IBLK), jnp.int32),
            pltpu.VMEM((2, IBLK, 128), jnp.float32),
            pltpu.VMEM((2, 8, 8, 128), jnp.float32),
            pltpu.SemaphoreType.DMA,
            pltpu.SemaphoreType.DMA,
            pltpu.SemaphoreType.DMA,
            pltpu.SemaphoreType.DMA,
        ],
        compiler_params=pltpu.CompilerParams(use_tc_tiling_on_sc=True,
                                             needs_layout_passes=False),
    )
    return fn(idx_t, tab_pairs)


def kernel(token_idxs, table):
    idx_t = token_idxs.T
    tab_pairs = table.reshape(VOCAB // 2, 128)
    out5 = _embed(idx_t, tab_pairs)
    return out5.transpose(2, 4, 0, 1, 3).reshape(ROWS, COLS, EMBED)


# speedup vs baseline: 1.4194x; 1.4194x over previous
import jax
import jax.numpy as jnp
from jax import lax
from jax.experimental import pallas as pl
from jax.experimental.pallas import tpu as pltpu
from jax.experimental.pallas import tpu_sc as plsc

ROWS, COLS = 4096, 200
EMBED = 64
VOCAB = 1000000
NC, NS = 2, 16
NW = NC * NS
IBLK = ROWS // NW
PITCH = 129


def _body(idx_hbm, tab_hbm, out_hbm, idx_v, pidx_v, rows_v, blk_v,
          gsem0, gsem1, osem0, osem1):
    w = lax.axis_index("s") * NC + lax.axis_index("c")
    gsems = (gsem0, gsem1)
    osems = (osem0, osem1)
    lanes = lax.iota(jnp.int32, 16)
    evecs = [lanes + 16 * t for t in range(EMBED // 16)]

    pltpu.sync_copy(idx_hbm.at[:, pl.ds(pl.multiple_of(w * IBLK, IBLK), IBLK)],
                    idx_v)

    def prep(j, b):
        for g in range(IBLK // 16):
            t = idx_v[j, pl.ds(16 * g, 16)]
            pidx_v[b, pl.ds(16 * g, 16)] = lax.shift_right_logical(t, 1)

    def start_gather(b):
        pltpu.async_copy(tab_hbm.at[pidx_v.at[b]], rows_v.at[b], gsems[b])

    prep(0, 0)
    start_gather(0)

    def step(i, carry):
        for b in range(2):
            j = i * 2 + b

            @pl.when(j < COLS - 1)
            def _():
                prep(j + 1, 1 - b)
                start_gather(1 - b)

            pltpu.make_async_copy(tab_hbm.at[pl.ds(0, IBLK)],
                                  rows_v.at[b], gsems[b]).wait()
            @pl.when(j >= 2)
            def _():
                pltpu.make_async_copy(blk_v.at[b, :, :, pl.ds(0, 128)],
                                      out_hbm.at[0, :, 0, :, :],
                                      osems[b]).wait()

            @plsc.parallel_loop(0, IBLK // 16)
            def _(g):
                qvec = (idx_v[j, pl.ds(g * 16, 16)] & 1) * EMBED
                for m in range(16):
                    q = qvec[m]
                    k = g * 16 + m
                    kv = jnp.broadcast_to(k, (16,)).astype(jnp.int32)
                    for t in range(EMBED // 16):
                        v = rows_v[b, k, pl.ds(q + 16 * t, 16)]
                        plsc.store_scatter(
                            blk_v.at[b],
                            [lax.shift_right_logical(evecs[t], 3),
                             evecs[t] & 7, kv], v)

            pltpu.async_copy(blk_v.at[b, :, :, pl.ds(0, 128)],
                             out_hbm.at[j, :, w, :, :], osems[b])
        return carry

    lax.fori_loop(0, COLS // 2, step, 0, unroll=False)

    for b in range(2):
        pltpu.make_async_copy(blk_v.at[b, :, :, pl.ds(0, 128)],
                              out_hbm.at[0, :, 0, :, :], osems[b]).wait()


@jax.jit
def _embed(idx_t, tab_pairs):
    mesh = plsc.VectorSubcoreMesh(core_axis_name="c", subcore_axis_name="s")
    fn = pl.kernel(
        _body,
        out_type=jax.ShapeDtypeStruct((COLS, 8, NW, 8, 128), jnp.float32),
        mesh=mesh,
        scratch_types=[
            pltpu.VMEM((COLS, IBLK), jnp.int32),
            pltpu.VMEM((2, IBLK), jnp.int32),
            pltpu.VMEM((2, IBLK, 128), jnp.float32),
            pltpu.VMEM((2, 8, 8, PITCH), jnp.float32),
            pltpu.SemaphoreType.DMA,
            pltpu.SemaphoreType.DMA,
            pltpu.SemaphoreType.DMA,
            pltpu.SemaphoreType.DMA,
        ],
        compiler_params=pltpu.CompilerParams(use_tc_tiling_on_sc=True,
                                             needs_layout_passes=False),
    )
    return fn(idx_t, tab_pairs)


def kernel(token_idxs, table):
    idx_t = token_idxs.T
    tab_pairs = table.reshape(VOCAB // 2, 128)
    out5 = _embed(idx_t, tab_pairs)
    return out5.transpose(2, 4, 0, 1, 3).reshape(ROWS, COLS, EMBED)
